# baseline (speedup 1.0000x reference)
import functools

import jax
import jax.numpy as jnp
from jax import lax
from jax.experimental import pallas as pl
from jax.experimental.pallas import tpu as pltpu
from jax.experimental.pallas import tpu_sc as plsc


def _stage_a(g2_ref, g1_ref, x1sq_ref, x2sq_ref, f1_ref, w1a_ref,
             h1a_ref, idx_ref, wgt_ref, *, n2):
    g2 = g2_ref[0]
    g1 = g1_ref[0]
    cross = jax.lax.dot_general(g2, g1, (((0,), (0,)), ((), ())),
                                preferred_element_type=jnp.float32)
    x1sq = x1sq_ref[0, 0, :]
    d2 = (x2sq_ref[0] + x1sq[None, :]) - 2.0 * cross

    iota = jax.lax.broadcasted_iota(jnp.int32, d2.shape, 0)
    cur = d2
    mins, amins = [], []
    for _ in range(3):
        m = jnp.min(cur, axis=0)
        am = jnp.min(jnp.where(cur == m[None, :], iota, n2), axis=0)
        mins.append(m)
        amins.append(am)
        cur = jnp.where(iota == am[None, :], jnp.float32(3e38), cur)

    ws = [1.0 / (jnp.sqrt(jnp.maximum(m, 0.0)) + 1e-8) for m in mins]
    norm = ws[0] + ws[1] + ws[2]
    ws = [w / norm for w in ws]

    base = pl.program_id(0) * n2
    gz = jnp.zeros_like(amins[0])
    idx_ref[0] = jnp.stack([amins[0] + base, amins[1] + base,
                            amins[2] + base, gz, gz, gz, gz, gz])
    fz = jnp.zeros_like(ws[0])
    wgt_ref[0] = jnp.stack([ws[0], ws[1], ws[2], fz, fz, fz, fz, fz])

    h1a_ref[0] = jnp.dot(w1a_ref[...], f1_ref[0],
                         preferred_element_type=jnp.float32)


def _make_sc_interp(P, C2, CH):
    info = plsc.get_sparse_core_info()
    NC, NS = info.num_cores, info.num_subcores
    NW = NC * NS
    per_w = P // NW
    n_chunks = per_w // CH
    mesh = plsc.VectorSubcoreMesh(core_axis_name="c", subcore_axis_name="s")

    @functools.partial(
        pl.kernel, mesh=mesh,
        out_type=jax.ShapeDtypeStruct((P, C2), jnp.float32),
        scratch_types=[
            pltpu.VMEM((CH,), jnp.int32),
            pltpu.VMEM((CH,), jnp.int32),
            pltpu.VMEM((CH,), jnp.int32),
            pltpu.VMEM((CH, 16), jnp.float32),
            pltpu.VMEM((CH, 16), jnp.float32),
            pltpu.VMEM((CH, 16), jnp.float32),
            pltpu.VMEM((CH, C2), jnp.float32),
            pltpu.VMEM((CH, C2), jnp.float32),
            pltpu.VMEM((CH, C2), jnp.float32),
            pltpu.VMEM((CH, C2), jnp.float32),
            pltpu.SemaphoreType.DMA,
            pltpu.SemaphoreType.DMA,
            pltpu.SemaphoreType.DMA,
        ],
    )
    def sc_interp(t_hbm, i0_hbm, i1_hbm, i2_hbm, w0_hbm, w1_hbm, w2_hbm,
                  out_hbm, iv0, iv1, iv2, wv0, wv1, wv2,
                  r0, r1, r2, acc, s0, s1, s2):
        wid = lax.axis_index("s") * NC + lax.axis_index("c")
        base = wid * per_w

        @pl.loop(0, n_chunks)
        def _chunk(c):
            off = base + c * CH
            pltpu.sync_copy(i0_hbm.at[pl.ds(off, CH)], iv0)
            pltpu.sync_copy(i1_hbm.at[pl.ds(off, CH)], iv1)
            pltpu.sync_copy(i2_hbm.at[pl.ds(off, CH)], iv2)
            pltpu.sync_copy(w0_hbm.at[pl.ds(off, CH)], wv0)
            pltpu.sync_copy(w1_hbm.at[pl.ds(off, CH)], wv1)
            pltpu.sync_copy(w2_hbm.at[pl.ds(off, CH)], wv2)
            c0 = pltpu.async_copy(t_hbm.at[iv0], r0, s0)
            c1 = pltpu.async_copy(t_hbm.at[iv1], r1, s1)
            c2 = pltpu.async_copy(t_hbm.at[iv2], r2, s2)
            c0.wait()
            c1.wait()
            c2.wait()

            @pl.loop(0, CH)
            def _point(i):
                wa = wv0[i, :]
                wb = wv1[i, :]
                wc = wv2[i, :]
                for j in range(C2 // 16):
                    s = pl.ds(j * 16, 16)
                    acc[i, s] = wa * r0[i, s] + wb * r1[i, s] + wc * r2[i, s]

            pltpu.sync_copy(acc, out_hbm.at[pl.ds(off, CH)])

    return sc_interp


def _stage_b(h1a_ref, itp_ref, w1b_ref, h1_ref, s_ref, q_ref):
    h1 = h1a_ref[0] + jax.lax.dot_general(
        w1b_ref[...], itp_ref[0], (((1,), (1,)), ((), ())),
        preferred_element_type=jnp.float32)
    h1_ref[0] = h1
    psum = jnp.sum(h1, axis=1, keepdims=True)
    psq = jnp.sum(h1 * h1, axis=1, keepdims=True)
    first = pl.program_id(1) == 0

    @pl.when(first)
    def _():
        s_ref[0] = psum
        q_ref[0] = psq

    @pl.when(jnp.logical_not(first))
    def _():
        s_ref[0] += psum
        q_ref[0] += psq


def _stage_c(h1_ref, s_ref, q_ref, gam_ref, bet_ref, w2_ref,
             h2_ref, s2_ref, q2_ref, *, count):
    mean = jnp.sum(s_ref[...], axis=0) / count
    var = jnp.sum(q_ref[...], axis=0) / count - mean * mean
    scale = gam_ref[...] * jax.lax.rsqrt(var + 1e-5)
    shift = bet_ref[...] - mean * scale
    act = jnp.maximum(scale * h1_ref[0] + shift, 0.0)
    h2 = jnp.dot(w2_ref[...], act, preferred_element_type=jnp.float32)
    h2_ref[0] = h2
    psum = jnp.sum(h2, axis=1, keepdims=True)
    psq = jnp.sum(h2 * h2, axis=1, keepdims=True)
    first = pl.program_id(1) == 0

    @pl.when(first)
    def _():
        s2_ref[0] = psum
        q2_ref[0] = psq

    @pl.when(jnp.logical_not(first))
    def _():
        s2_ref[0] += psum
        q2_ref[0] += psq


def _stage_d(h2_ref, s_ref, q_ref, gam_ref, bet_ref, out_ref, *, count):
    mean = jnp.sum(s_ref[...], axis=0) / count
    var = jnp.sum(q_ref[...], axis=0) / count - mean * mean
    scale = gam_ref[...] * jax.lax.rsqrt(var + 1e-5)
    shift = bet_ref[...] - mean * scale
    out_ref[0] = jnp.maximum(scale * h2_ref[0] + shift, 0.0)


def kernel(xyz1, xyz2, features1, features2, W1, g1, b1, W2, g2, b2):
    B, N1, _ = xyz1.shape
    N2 = xyz2.shape[1]
    C1 = features1.shape[1]
    C2 = features2.shape[1]
    CO1 = W1.shape[0]
    CO2 = W2.shape[0]
    TN = 256
    NT = N1 // TN
    P = B * N1
    f32 = jnp.float32

    bf16 = jnp.bfloat16
    x1t = jnp.transpose(xyz1, (0, 2, 1)).astype(bf16)
    x2t = jnp.transpose(xyz2, (0, 2, 1)).astype(bf16)
    x1sq = jnp.sum(xyz1 ** 2, axis=-1)
    x2sq = jnp.sum(xyz2 ** 2, axis=-1)
    G1 = jnp.concatenate([x1t, jnp.zeros((B, 13, N1), bf16)], axis=1)
    G2 = jnp.concatenate([x2t, jnp.zeros((B, 13, N2), bf16)], axis=1)
    X1SQ = x1sq[:, None, :]
    X2SQ = x2sq[:, :, None]

    W1a = W1[:, :C1]
    W1b = W1[:, C1:]

    grid = (B, NT)
    params = pltpu.CompilerParams(
        dimension_semantics=("parallel", "arbitrary"))
    h1a, IDX, WGT = pl.pallas_call(
        functools.partial(_stage_a, n2=N2),
        grid=grid,
        in_specs=[
            pl.BlockSpec((1, 16, N2), lambda b, t: (b, 0, 0)),
            pl.BlockSpec((1, 16, TN), lambda b, t: (b, 0, t)),
            pl.BlockSpec((1, 1, TN), lambda b, t: (b, 0, t)),
            pl.BlockSpec((1, N2, 1), lambda b, t: (b, 0, 0)),
            pl.BlockSpec((1, C1, TN), lambda b, t: (b, 0, t)),
            pl.BlockSpec((CO1, C1), lambda b, t: (0, 0)),
        ],
        out_specs=[
            pl.BlockSpec((1, CO1, TN), lambda b, t: (b, 0, t)),
            pl.BlockSpec((1, 8, TN), lambda b, t: (b, 0, t)),
            pl.BlockSpec((1, 8, TN), lambda b, t: (b, 0, t)),
        ],
        out_shape=[
            jax.ShapeDtypeStruct((B, CO1, N1), f32),
            jax.ShapeDtypeStruct((B, 8, N1), jnp.int32),
            jax.ShapeDtypeStruct((B, 8, N1), f32),
        ],
        compiler_params=params,
    )(G2, G1, X1SQ, X2SQ, features1, W1a)

    table = jnp.transpose(features2, (0, 2, 1)).reshape(B * N2, C2)
    idx0 = IDX[:, 0, :].reshape(P)
    idx1 = IDX[:, 1, :].reshape(P)
    idx2 = IDX[:, 2, :].reshape(P)
    wb0 = jnp.broadcast_to(WGT[:, 0, :].reshape(P)[:, None], (P, 16))
    wb1 = jnp.broadcast_to(WGT[:, 1, :].reshape(P)[:, None], (P, 16))
    wb2 = jnp.broadcast_to(WGT[:, 2, :].reshape(P)[:, None], (P, 16))

    sc_interp = _make_sc_interp(P, C2, 64)
    interp = sc_interp(table, idx0, idx1, idx2, wb0, wb1, wb2)
    interp = interp.reshape(B, N1, C2)

    count = float(B * N1)
    h1, s1, q1 = pl.pallas_call(
        _stage_b,
        grid=grid,
        in_specs=[
            pl.BlockSpec((1, CO1, TN), lambda b, t: (b, 0, t)),
            pl.BlockSpec((1, TN, C2), lambda b, t: (b, t, 0)),
            pl.BlockSpec((CO1, C2), lambda b, t: (0, 0)),
        ],
        out_specs=[
            pl.BlockSpec((1, CO1, TN), lambda b, t: (b, 0, t)),
            pl.BlockSpec((1, CO1, 1), lambda b, t: (b, 0, 0)),
            pl.BlockSpec((1, CO1, 1), lambda b, t: (b, 0, 0)),
        ],
        out_shape=[
            jax.ShapeDtypeStruct((B, CO1, N1), f32),
            jax.ShapeDtypeStruct((B, CO1, 1), f32),
            jax.ShapeDtypeStruct((B, CO1, 1), f32),
        ],
        compiler_params=params,
    )(h1a, interp, W1b)

    h2, s2, q2 = pl.pallas_call(
        functools.partial(_stage_c, count=count),
        grid=grid,
        in_specs=[
            pl.BlockSpec((1, CO1, TN), lambda b, t: (b, 0, t)),
            pl.BlockSpec((B, CO1, 1), lambda b, t: (0, 0, 0)),
            pl.BlockSpec((B, CO1, 1), lambda b, t: (0, 0, 0)),
            pl.BlockSpec((CO1, 1), lambda b, t: (0, 0)),
            pl.BlockSpec((CO1, 1), lambda b, t: (0, 0)),
            pl.BlockSpec((CO2, CO1), lambda b, t: (0, 0)),
        ],
        out_specs=[
            pl.BlockSpec((1, CO2, TN), lambda b, t: (b, 0, t)),
            pl.BlockSpec((1, CO2, 1), lambda b, t: (b, 0, 0)),
            pl.BlockSpec((1, CO2, 1), lambda b, t: (b, 0, 0)),
        ],
        out_shape=[
            jax.ShapeDtypeStruct((B, CO2, N1), f32),
            jax.ShapeDtypeStruct((B, CO2, 1), f32),
            jax.ShapeDtypeStruct((B, CO2, 1), f32),
        ],
        compiler_params=params,
    )(h1, s1, q1, g1.reshape(CO1, 1), b1.reshape(CO1, 1), W2)

    out = pl.pallas_call(
        functools.partial(_stage_d, count=count),
        grid=grid,
        in_specs=[
            pl.BlockSpec((1, CO2, TN), lambda b, t: (b, 0, t)),
            pl.BlockSpec((B, CO2, 1), lambda b, t: (0, 0, 0)),
            pl.BlockSpec((B, CO2, 1), lambda b, t: (0, 0, 0)),
            pl.BlockSpec((CO2, 1), lambda b, t: (0, 0)),
            pl.BlockSpec((CO2, 1), lambda b, t: (0, 0)),
        ],
        out_specs=pl.BlockSpec((1, CO2, TN), lambda b, t: (b, 0, t)),
        out_shape=jax.ShapeDtypeStruct((B, CO2, N1), f32),
        compiler_params=params,
    )(h2, s2, q2, g2.reshape(CO2, 1), b2.reshape(CO2, 1))

    return out


# speedup vs baseline: 2.0566x; 2.0566x over previous
import functools

import jax
import jax.numpy as jnp
from jax.experimental import pallas as pl
from jax.experimental.pallas import tpu as pltpu


def _stage_a(g2_ref, g1_ref, x1sq_ref, x2sq_ref, f1_ref, f2_ref,
             w1a_ref, w1b_ref, h1_ref, s_ref, q_ref, *, n2):
    g2 = g2_ref[0]
    g1 = g1_ref[0]
    cross = jax.lax.dot_general(g2, g1, (((0,), (0,)), ((), ())),
                                preferred_element_type=jnp.float32)
    x1sq = x1sq_ref[0, 0, :]
    d2 = (x2sq_ref[0] + x1sq[None, :]) - 2.0 * cross

    iota = jax.lax.broadcasted_iota(jnp.int32, d2.shape, 0)
    cur = d2
    mins, amins = [], []
    for _ in range(3):
        m = jnp.min(cur, axis=0)
        am = jnp.min(jnp.where(cur == m[None, :], iota, n2), axis=0)
        mins.append(m)
        amins.append(am)
        cur = jnp.where(iota == am[None, :], jnp.float32(3e38), cur)

    ws = [1.0 / (jnp.sqrt(jnp.maximum(m, 0.0)) + 1e-8) for m in mins]
    norm = ws[0] + ws[1] + ws[2]
    ws = [w / norm for w in ws]

    st = jnp.where(iota == amins[0][None, :], ws[0][None, :], 0.0)
    st = st + jnp.where(iota == amins[1][None, :], ws[1][None, :], 0.0)
    st = st + jnp.where(iota == amins[2][None, :], ws[2][None, :], 0.0)

    interp = jnp.dot(f2_ref[0], st, preferred_element_type=jnp.float32)
    h1 = (jnp.dot(w1a_ref[...], f1_ref[0], preferred_element_type=jnp.float32)
          + jnp.dot(w1b_ref[...], interp, preferred_element_type=jnp.float32))
    h1_ref[0] = h1

    psum = jnp.sum(h1, axis=1, keepdims=True)
    psq = jnp.sum(h1 * h1, axis=1, keepdims=True)
    first = pl.program_id(1) == 0

    @pl.when(first)
    def _():
        s_ref[0] = psum
        q_ref[0] = psq

    @pl.when(jnp.logical_not(first))
    def _():
        s_ref[0] += psum
        q_ref[0] += psq


def _stage_b(h1_ref, s_ref, q_ref, gam_ref, bet_ref, w2_ref,
             h2_ref, s2_ref, q2_ref, *, count):
    mean = jnp.sum(s_ref[...], axis=0) / count
    var = jnp.sum(q_ref[...], axis=0) / count - mean * mean
    scale = gam_ref[...] * jax.lax.rsqrt(var + 1e-5)
    shift = bet_ref[...] - mean * scale
    act = jnp.maximum(scale * h1_ref[0] + shift, 0.0)
    h2 = jnp.dot(w2_ref[...], act, preferred_element_type=jnp.float32)
    h2_ref[0] = h2

    psum = jnp.sum(h2, axis=1, keepdims=True)
    psq = jnp.sum(h2 * h2, axis=1, keepdims=True)
    first = pl.program_id(1) == 0

    @pl.when(first)
    def _():
        s2_ref[0] = psum
        q2_ref[0] = psq

    @pl.when(jnp.logical_not(first))
    def _():
        s2_ref[0] += psum
        q2_ref[0] += psq


def _stage_c(h2_ref, s_ref, q_ref, gam_ref, bet_ref, out_ref, *, count):
    mean = jnp.sum(s_ref[...], axis=0) / count
    var = jnp.sum(q_ref[...], axis=0) / count - mean * mean
    scale = gam_ref[...] * jax.lax.rsqrt(var + 1e-5)
    shift = bet_ref[...] - mean * scale
    out_ref[0] = jnp.maximum(scale * h2_ref[0] + shift, 0.0)


def kernel(xyz1, xyz2, features1, features2, W1, g1, b1, W2, g2, b2):
    B, N1, _ = xyz1.shape
    N2 = xyz2.shape[1]
    C1 = features1.shape[1]
    C2 = features2.shape[1]
    CO1 = W1.shape[0]
    CO2 = W2.shape[0]
    TN = 512
    NT = N1 // TN
    f32 = jnp.float32

    bf16 = jnp.bfloat16
    x1t = jnp.transpose(xyz1, (0, 2, 1)).astype(bf16)
    x2t = jnp.transpose(xyz2, (0, 2, 1)).astype(bf16)
    x1sq = jnp.sum(xyz1 ** 2, axis=-1)
    x2sq = jnp.sum(xyz2 ** 2, axis=-1)
    zer1 = jnp.zeros((B, 13, N1), bf16)
    zer2 = jnp.zeros((B, 13, N2), bf16)
    G1 = jnp.concatenate([x1t, zer1], axis=1)
    G2 = jnp.concatenate([x2t, zer2], axis=1)
    X1SQ = x1sq[:, None, :]
    X2SQ = x2sq[:, :, None]

    W1a = W1[:, :C1]
    W1b = W1[:, C1:]

    grid = (B, NT)
    params = pltpu.CompilerParams(
        dimension_semantics=("parallel", "arbitrary"))
    h1, s1, q1 = pl.pallas_call(
        functools.partial(_stage_a, n2=N2),
        grid=grid,
        in_specs=[
            pl.BlockSpec((1, 16, N2), lambda b, t: (b, 0, 0)),
            pl.BlockSpec((1, 16, TN), lambda b, t: (b, 0, t)),
            pl.BlockSpec((1, 1, TN), lambda b, t: (b, 0, t)),
            pl.BlockSpec((1, N2, 1), lambda b, t: (b, 0, 0)),
            pl.BlockSpec((1, C1, TN), lambda b, t: (b, 0, t)),
            pl.BlockSpec((1, C2, N2), lambda b, t: (b, 0, 0)),
            pl.BlockSpec((CO1, C1), lambda b, t: (0, 0)),
            pl.BlockSpec((CO1, C2), lambda b, t: (0, 0)),
        ],
        out_specs=[
            pl.BlockSpec((1, CO1, TN), lambda b, t: (b, 0, t)),
            pl.BlockSpec((1, CO1, 1), lambda b, t: (b, 0, 0)),
            pl.BlockSpec((1, CO1, 1), lambda b, t: (b, 0, 0)),
        ],
        out_shape=[
            jax.ShapeDtypeStruct((B, CO1, N1), f32),
            jax.ShapeDtypeStruct((B, CO1, 1), f32),
            jax.ShapeDtypeStruct((B, CO1, 1), f32),
        ],
        compiler_params=params,
    )(G2, G1, X1SQ, X2SQ, features1, features2, W1a, W1b)

    count = float(B * N1)
    h2, s2, q2 = pl.pallas_call(
        functools.partial(_stage_b, count=count),
        grid=grid,
        in_specs=[
            pl.BlockSpec((1, CO1, TN), lambda b, t: (b, 0, t)),
            pl.BlockSpec((B, CO1, 1), lambda b, t: (0, 0, 0)),
            pl.BlockSpec((B, CO1, 1), lambda b, t: (0, 0, 0)),
            pl.BlockSpec((CO1, 1), lambda b, t: (0, 0)),
            pl.BlockSpec((CO1, 1), lambda b, t: (0, 0)),
            pl.BlockSpec((CO2, CO1), lambda b, t: (0, 0)),
        ],
        out_specs=[
            pl.BlockSpec((1, CO2, TN), lambda b, t: (b, 0, t)),
            pl.BlockSpec((1, CO2, 1), lambda b, t: (b, 0, 0)),
            pl.BlockSpec((1, CO2, 1), lambda b, t: (b, 0, 0)),
        ],
        out_shape=[
            jax.ShapeDtypeStruct((B, CO2, N1), f32),
            jax.ShapeDtypeStruct((B, CO2, 1), f32),
            jax.ShapeDtypeStruct((B, CO2, 1), f32),
        ],
        compiler_params=params,
    )(h1, s1, q1, g1.reshape(CO1, 1), b1.reshape(CO1, 1), W2)

    out = pl.pallas_call(
        functools.partial(_stage_c, count=count),
        grid=grid,
        in_specs=[
            pl.BlockSpec((1, CO2, TN), lambda b, t: (b, 0, t)),
            pl.BlockSpec((B, CO2, 1), lambda b, t: (0, 0, 0)),
            pl.BlockSpec((B, CO2, 1), lambda b, t: (0, 0, 0)),
            pl.BlockSpec((CO2, 1), lambda b, t: (0, 0)),
            pl.BlockSpec((CO2, 1), lambda b, t: (0, 0)),
        ],
        out_specs=pl.BlockSpec((1, CO2, TN), lambda b, t: (b, 0, t)),
        out_shape=jax.ShapeDtypeStruct((B, CO2, N1), f32),
        compiler_params=params,
    )(h2, s2, q2, g2.reshape(CO2, 1), b2.reshape(CO2, 1))

    return out


# speedup vs baseline: 2.6021x; 1.2652x over previous
import functools

import jax
import jax.numpy as jnp
from jax.experimental import pallas as pl
from jax.experimental.pallas import tpu as pltpu


def _stage_a(g2_ref, g1_ref, x1sq_ref, x2sq_ref, f1_ref, f2_ref,
             w1a_ref, w1b_ref, h1_ref, s_ref, q_ref, *, n2):
    g2 = g2_ref[0]
    g1 = g1_ref[0]
    cross = jax.lax.dot_general(g2, g1, (((0,), (0,)), ((), ())),
                                preferred_element_type=jnp.float32)
    x1sq = x1sq_ref[0, 0, :]
    d2 = (x2sq_ref[0] + x1sq[None, :]) - 2.0 * cross

    iota = jax.lax.broadcasted_iota(jnp.int32, d2.shape, 0)
    cur = d2
    mins, amins = [], []
    for _ in range(3):
        m = jnp.min(cur, axis=0)
        am = jnp.min(jnp.where(cur == m[None, :], iota, n2), axis=0)
        mins.append(m)
        amins.append(am)
        cur = jnp.where(iota == am[None, :], jnp.float32(3e38), cur)

    ws = [1.0 / (jnp.sqrt(jnp.maximum(m, 0.0)) + 1e-8) for m in mins]
    norm = ws[0] + ws[1] + ws[2]
    ws = [w / norm for w in ws]

    st = jnp.where(iota == amins[0][None, :], ws[0][None, :], 0.0)
    st = st + jnp.where(iota == amins[1][None, :], ws[1][None, :], 0.0)
    st = st + jnp.where(iota == amins[2][None, :], ws[2][None, :], 0.0)

    interp = jnp.dot(f2_ref[0], st, preferred_element_type=jnp.float32)
    h1 = (jnp.dot(w1a_ref[...], f1_ref[0], preferred_element_type=jnp.float32)
          + jnp.dot(w1b_ref[...], interp, preferred_element_type=jnp.float32))
    h1_ref[0] = h1

    psum = jnp.sum(h1, axis=1, keepdims=True)
    psq = jnp.sum(h1 * h1, axis=1, keepdims=True)
    first = pl.program_id(1) == 0

    @pl.when(first)
    def _():
        s_ref[0] = psum
        q_ref[0] = psq

    @pl.when(jnp.logical_not(first))
    def _():
        s_ref[0] += psum
        q_ref[0] += psq


def _stage_b(h1_ref, s_ref, q_ref, gam_ref, bet_ref, w2_ref,
             h2_ref, s2_ref, q2_ref, *, count):
    mean = jnp.sum(s_ref[...], axis=0) / count
    var = jnp.sum(q_ref[...], axis=0) / count - mean * mean
    scale = gam_ref[...] * jax.lax.rsqrt(var + 1e-5)
    shift = bet_ref[...] - mean * scale
    act = jnp.maximum(scale * h1_ref[0] + shift, 0.0)
    h2 = jnp.dot(w2_ref[...], act, preferred_element_type=jnp.float32)
    h2_ref[0] = h2

    psum = jnp.sum(h2, axis=1, keepdims=True)
    psq = jnp.sum(h2 * h2, axis=1, keepdims=True)
    first = pl.program_id(1) == 0

    @pl.when(first)
    def _():
        s2_ref[0] = psum
        q2_ref[0] = psq

    @pl.when(jnp.logical_not(first))
    def _():
        s2_ref[0] += psum
        q2_ref[0] += psq


def _stage_c(h2_ref, s_ref, q_ref, gam_ref, bet_ref, out_ref, *, count):
    mean = jnp.sum(s_ref[...], axis=0) / count
    var = jnp.sum(q_ref[...], axis=0) / count - mean * mean
    scale = gam_ref[...] * jax.lax.rsqrt(var + 1e-5)
    shift = bet_ref[...] - mean * scale
    out_ref[0] = jnp.maximum(scale * h2_ref[0] + shift, 0.0)


def kernel(xyz1, xyz2, features1, features2, W1, g1, b1, W2, g2, b2):
    B, N1, _ = xyz1.shape
    N2 = xyz2.shape[1]
    C1 = features1.shape[1]
    C2 = features2.shape[1]
    CO1 = W1.shape[0]
    CO2 = W2.shape[0]
    TN = 1024
    NT = N1 // TN
    f32 = jnp.float32

    bf16 = jnp.bfloat16
    x1t = jnp.transpose(xyz1, (0, 2, 1)).astype(bf16)
    x2t = jnp.transpose(xyz2, (0, 2, 1)).astype(bf16)
    x1sq = jnp.sum(xyz1 ** 2, axis=-1)
    x2sq = jnp.sum(xyz2 ** 2, axis=-1)
    zer1 = jnp.zeros((B, 13, N1), bf16)
    zer2 = jnp.zeros((B, 13, N2), bf16)
    G1 = jnp.concatenate([x1t, zer1], axis=1)
    G2 = jnp.concatenate([x2t, zer2], axis=1)
    X1SQ = x1sq[:, None, :]
    X2SQ = x2sq[:, :, None]

    W1a = W1[:, :C1]
    W1b = W1[:, C1:]

    grid = (B, NT)
    params = pltpu.CompilerParams(
        dimension_semantics=("parallel", "arbitrary"))
    h1, s1, q1 = pl.pallas_call(
        functools.partial(_stage_a, n2=N2),
        grid=grid,
        in_specs=[
            pl.BlockSpec((1, 16, N2), lambda b, t: (b, 0, 0)),
            pl.BlockSpec((1, 16, TN), lambda b, t: (b, 0, t)),
            pl.BlockSpec((1, 1, TN), lambda b, t: (b, 0, t)),
            pl.BlockSpec((1, N2, 1), lambda b, t: (b, 0, 0)),
            pl.BlockSpec((1, C1, TN), lambda b, t: (b, 0, t)),
            pl.BlockSpec((1, C2, N2), lambda b, t: (b, 0, 0)),
            pl.BlockSpec((CO1, C1), lambda b, t: (0, 0)),
            pl.BlockSpec((CO1, C2), lambda b, t: (0, 0)),
        ],
        out_specs=[
            pl.BlockSpec((1, CO1, TN), lambda b, t: (b, 0, t)),
            pl.BlockSpec((1, CO1, 1), lambda b, t: (b, 0, 0)),
            pl.BlockSpec((1, CO1, 1), lambda b, t: (b, 0, 0)),
        ],
        out_shape=[
            jax.ShapeDtypeStruct((B, CO1, N1), f32),
            jax.ShapeDtypeStruct((B, CO1, 1), f32),
            jax.ShapeDtypeStruct((B, CO1, 1), f32),
        ],
        compiler_params=params,
    )(G2, G1, X1SQ, X2SQ, features1, features2, W1a, W1b)

    count = float(B * N1)
    h2, s2, q2 = pl.pallas_call(
        functools.partial(_stage_b, count=count),
        grid=grid,
        in_specs=[
            pl.BlockSpec((1, CO1, TN), lambda b, t: (b, 0, t)),
            pl.BlockSpec((B, CO1, 1), lambda b, t: (0, 0, 0)),
            pl.BlockSpec((B, CO1, 1), lambda b, t: (0, 0, 0)),
            pl.BlockSpec((CO1, 1), lambda b, t: (0, 0)),
            pl.BlockSpec((CO1, 1), lambda b, t: (0, 0)),
            pl.BlockSpec((CO2, CO1), lambda b, t: (0, 0)),
        ],
        out_specs=[
            pl.BlockSpec((1, CO2, TN), lambda b, t: (b, 0, t)),
            pl.BlockSpec((1, CO2, 1), lambda b, t: (b, 0, 0)),
            pl.BlockSpec((1, CO2, 1), lambda b, t: (b, 0, 0)),
        ],
        out_shape=[
            jax.ShapeDtypeStruct((B, CO2, N1), f32),
            jax.ShapeDtypeStruct((B, CO2, 1), f32),
            jax.ShapeDtypeStruct((B, CO2, 1), f32),
        ],
        compiler_params=params,
    )(h1, s1, q1, g1.reshape(CO1, 1), b1.reshape(CO1, 1), W2)

    out = pl.pallas_call(
        functools.partial(_stage_c, count=count),
        grid=grid,
        in_specs=[
            pl.BlockSpec((1, CO2, TN), lambda b, t: (b, 0, t)),
            pl.BlockSpec((B, CO2, 1), lambda b, t: (0, 0, 0)),
            pl.BlockSpec((B, CO2, 1), lambda b, t: (0, 0, 0)),
            pl.BlockSpec((CO2, 1), lambda b, t: (0, 0)),
            pl.BlockSpec((CO2, 1), lambda b, t: (0, 0)),
        ],
        out_specs=pl.BlockSpec((1, CO2, TN), lambda b, t: (b, 0, t)),
        out_shape=jax.ShapeDtypeStruct((B, CO2, N1), f32),
        compiler_params=params,
    )(h2, s2, q2, g2.reshape(CO2, 1), b2.reshape(CO2, 1))

    return out


# speedup vs baseline: 2.9477x; 1.1329x over previous
import functools

import jax
import jax.numpy as jnp
from jax.experimental import pallas as pl
from jax.experimental.pallas import tpu as pltpu


def _stage_a(g2_ref, g1_ref, x1sq_ref, x2sq_ref, f1_ref, f2_ref,
             w1a_ref, w1b_ref, h1_ref, s_ref, q_ref, *, n2):
    g2 = g2_ref[0]
    g1 = g1_ref[0]
    cross = jax.lax.dot_general(g2, g1, (((0,), (0,)), ((), ())),
                                preferred_element_type=jnp.float32)
    x1sq = x1sq_ref[0, 0, :]
    d2 = (x2sq_ref[0] + x1sq[None, :]) - 2.0 * cross

    iota = jax.lax.broadcasted_iota(jnp.int32, d2.shape, 0)
    cur = d2
    mins, amins = [], []
    for _ in range(3):
        m = jnp.min(cur, axis=0)
        am = jnp.min(jnp.where(cur == m[None, :], iota, n2), axis=0)
        mins.append(m)
        amins.append(am)
        cur = jnp.where(iota == am[None, :], jnp.float32(3e38), cur)

    ws = [1.0 / (jnp.sqrt(jnp.maximum(m, 0.0)) + 1e-8) for m in mins]
    norm = ws[0] + ws[1] + ws[2]
    ws = [w / norm for w in ws]

    st = jnp.where(iota == amins[0][None, :], ws[0][None, :], 0.0)
    st = st + jnp.where(iota == amins[1][None, :], ws[1][None, :], 0.0)
    st = st + jnp.where(iota == amins[2][None, :], ws[2][None, :], 0.0)

    interp = jnp.dot(f2_ref[0], st, preferred_element_type=jnp.float32)
    h1 = (jnp.dot(w1a_ref[...], f1_ref[0], preferred_element_type=jnp.float32)
          + jnp.dot(w1b_ref[...], interp, preferred_element_type=jnp.float32))
    h1_ref[0] = h1

    psum = jnp.sum(h1, axis=1, keepdims=True)
    psq = jnp.sum(h1 * h1, axis=1, keepdims=True)
    first = pl.program_id(1) == 0

    @pl.when(first)
    def _():
        s_ref[0] = psum
        q_ref[0] = psq

    @pl.when(jnp.logical_not(first))
    def _():
        s_ref[0] += psum
        q_ref[0] += psq


def _stage_b(h1_ref, s_ref, q_ref, gam_ref, bet_ref, w2_ref,
             h2_ref, s2_ref, q2_ref, *, count):
    mean = jnp.sum(s_ref[...], axis=0) / count
    var = jnp.sum(q_ref[...], axis=0) / count - mean * mean
    scale = gam_ref[...] * jax.lax.rsqrt(var + 1e-5)
    shift = bet_ref[...] - mean * scale
    act = jnp.maximum(scale * h1_ref[0] + shift, 0.0)
    h2 = jnp.dot(w2_ref[...], act, preferred_element_type=jnp.float32)
    h2_ref[0] = h2

    psum = jnp.sum(h2, axis=1, keepdims=True)
    psq = jnp.sum(h2 * h2, axis=1, keepdims=True)
    first = pl.program_id(1) == 0

    @pl.when(first)
    def _():
        s2_ref[0] = psum
        q2_ref[0] = psq

    @pl.when(jnp.logical_not(first))
    def _():
        s2_ref[0] += psum
        q2_ref[0] += psq


def _stage_c(h2_ref, s_ref, q_ref, gam_ref, bet_ref, out_ref, *, count):
    mean = jnp.sum(s_ref[...], axis=0) / count
    var = jnp.sum(q_ref[...], axis=0) / count - mean * mean
    scale = gam_ref[...] * jax.lax.rsqrt(var + 1e-5)
    shift = bet_ref[...] - mean * scale
    out_ref[0] = jnp.maximum(scale * h2_ref[0] + shift, 0.0)


def kernel(xyz1, xyz2, features1, features2, W1, g1, b1, W2, g2, b2):
    B, N1, _ = xyz1.shape
    N2 = xyz2.shape[1]
    C1 = features1.shape[1]
    C2 = features2.shape[1]
    CO1 = W1.shape[0]
    CO2 = W2.shape[0]
    TN = 2048
    NT = N1 // TN
    f32 = jnp.float32

    bf16 = jnp.bfloat16
    x1t = jnp.transpose(xyz1, (0, 2, 1)).astype(bf16)
    x2t = jnp.transpose(xyz2, (0, 2, 1)).astype(bf16)
    x1sq = jnp.sum(xyz1 ** 2, axis=-1)
    x2sq = jnp.sum(xyz2 ** 2, axis=-1)
    zer1 = jnp.zeros((B, 13, N1), bf16)
    zer2 = jnp.zeros((B, 13, N2), bf16)
    G1 = jnp.concatenate([x1t, zer1], axis=1)
    G2 = jnp.concatenate([x2t, zer2], axis=1)
    X1SQ = x1sq[:, None, :]
    X2SQ = x2sq[:, :, None]

    W1a = W1[:, :C1]
    W1b = W1[:, C1:]

    grid = (B, NT)
    params = pltpu.CompilerParams(
        dimension_semantics=("parallel", "arbitrary"))
    h1, s1, q1 = pl.pallas_call(
        functools.partial(_stage_a, n2=N2),
        grid=grid,
        in_specs=[
            pl.BlockSpec((1, 16, N2), lambda b, t: (b, 0, 0)),
            pl.BlockSpec((1, 16, TN), lambda b, t: (b, 0, t)),
            pl.BlockSpec((1, 1, TN), lambda b, t: (b, 0, t)),
            pl.BlockSpec((1, N2, 1), lambda b, t: (b, 0, 0)),
            pl.BlockSpec((1, C1, TN), lambda b, t: (b, 0, t)),
            pl.BlockSpec((1, C2, N2), lambda b, t: (b, 0, 0)),
            pl.BlockSpec((CO1, C1), lambda b, t: (0, 0)),
            pl.BlockSpec((CO1, C2), lambda b, t: (0, 0)),
        ],
        out_specs=[
            pl.BlockSpec((1, CO1, TN), lambda b, t: (b, 0, t)),
            pl.BlockSpec((1, CO1, 1), lambda b, t: (b, 0, 0)),
            pl.BlockSpec((1, CO1, 1), lambda b, t: (b, 0, 0)),
        ],
        out_shape=[
            jax.ShapeDtypeStruct((B, CO1, N1), f32),
            jax.ShapeDtypeStruct((B, CO1, 1), f32),
            jax.ShapeDtypeStruct((B, CO1, 1), f32),
        ],
        compiler_params=params,
    )(G2, G1, X1SQ, X2SQ, features1, features2, W1a, W1b)

    count = float(B * N1)
    h2, s2, q2 = pl.pallas_call(
        functools.partial(_stage_b, count=count),
        grid=grid,
        in_specs=[
            pl.BlockSpec((1, CO1, TN), lambda b, t: (b, 0, t)),
            pl.BlockSpec((B, CO1, 1), lambda b, t: (0, 0, 0)),
            pl.BlockSpec((B, CO1, 1), lambda b, t: (0, 0, 0)),
            pl.BlockSpec((CO1, 1), lambda b, t: (0, 0)),
            pl.BlockSpec((CO1, 1), lambda b, t: (0, 0)),
            pl.BlockSpec((CO2, CO1), lambda b, t: (0, 0)),
        ],
        out_specs=[
            pl.BlockSpec((1, CO2, TN), lambda b, t: (b, 0, t)),
            pl.BlockSpec((1, CO2, 1), lambda b, t: (b, 0, 0)),
            pl.BlockSpec((1, CO2, 1), lambda b, t: (b, 0, 0)),
        ],
        out_shape=[
            jax.ShapeDtypeStruct((B, CO2, N1), f32),
            jax.ShapeDtypeStruct((B, CO2, 1), f32),
            jax.ShapeDtypeStruct((B, CO2, 1), f32),
        ],
        compiler_params=params,
    )(h1, s1, q1, g1.reshape(CO1, 1), b1.reshape(CO1, 1), W2)

    out = pl.pallas_call(
        functools.partial(_stage_c, count=count),
        grid=grid,
        in_specs=[
            pl.BlockSpec((1, CO2, TN), lambda b, t: (b, 0, t)),
            pl.BlockSpec((B, CO2, 1), lambda b, t: (0, 0, 0)),
            pl.BlockSpec((B, CO2, 1), lambda b, t: (0, 0, 0)),
            pl.BlockSpec((CO2, 1), lambda b, t: (0, 0)),
            pl.BlockSpec((CO2, 1), lambda b, t: (0, 0)),
        ],
        out_specs=pl.BlockSpec((1, CO2, TN), lambda b, t: (b, 0, t)),
        out_shape=jax.ShapeDtypeStruct((B, CO2, N1), f32),
        compiler_params=params,
    )(h2, s2, q2, g2.reshape(CO2, 1), b2.reshape(CO2, 1))

    return out


# speedup vs baseline: 3.0562x; 1.0368x over previous
import functools

import jax
import jax.numpy as jnp
from jax.experimental import pallas as pl
from jax.experimental.pallas import tpu as pltpu


def _stage_a(g2_ref, g1_ref, x1sq_ref, x2sq_ref, f1_ref, f2_ref,
             w1a_ref, w1b_ref, h1_ref, s_ref, q_ref, *, n2):
    g2 = g2_ref[0]
    g1 = g1_ref[0]
    cross = jax.lax.dot_general(g2, g1, (((0,), (0,)), ((), ())),
                                preferred_element_type=jnp.float32)
    x1sq = x1sq_ref[0, 0, :]
    d2 = (x2sq_ref[0] + x1sq[None, :]) - 2.0 * cross

    iota = jax.lax.broadcasted_iota(jnp.int32, d2.shape, 0)
    cur = d2
    mins, amins = [], []
    for _ in range(3):
        m = jnp.min(cur, axis=0)
        am = jnp.min(jnp.where(cur == m[None, :], iota, n2), axis=0)
        mins.append(m)
        amins.append(am)
        cur = jnp.where(iota == am[None, :], jnp.float32(3e38), cur)

    ws = [1.0 / (jnp.sqrt(jnp.maximum(m, 0.0)) + 1e-8) for m in mins]
    norm = ws[0] + ws[1] + ws[2]
    ws = [w / norm for w in ws]

    st = jnp.where(iota == amins[0][None, :], ws[0][None, :], 0.0)
    st = st + jnp.where(iota == amins[1][None, :], ws[1][None, :], 0.0)
    st = st + jnp.where(iota == amins[2][None, :], ws[2][None, :], 0.0)

    interp = jnp.dot(f2_ref[0], st, preferred_element_type=jnp.float32)
    h1 = (jnp.dot(w1a_ref[...], f1_ref[0], preferred_element_type=jnp.float32)
          + jnp.dot(w1b_ref[...], interp, preferred_element_type=jnp.float32))
    h1_ref[0] = h1

    psum = jnp.sum(h1, axis=1, keepdims=True)
    psq = jnp.sum(h1 * h1, axis=1, keepdims=True)
    first = pl.program_id(1) == 0

    @pl.when(first)
    def _():
        s_ref[0] = psum
        q_ref[0] = psq

    @pl.when(jnp.logical_not(first))
    def _():
        s_ref[0] += psum
        q_ref[0] += psq


def _stage_b(h1_ref, s_ref, q_ref, gam_ref, bet_ref, w2_ref,
             h2_ref, s2_ref, q2_ref, *, count):
    mean = jnp.sum(s_ref[...], axis=0) / count
    var = jnp.sum(q_ref[...], axis=0) / count - mean * mean
    scale = gam_ref[...] * jax.lax.rsqrt(var + 1e-5)
    shift = bet_ref[...] - mean * scale
    act = jnp.maximum(scale * h1_ref[0] + shift, 0.0)
    h2 = jnp.dot(w2_ref[...], act, preferred_element_type=jnp.float32)
    h2_ref[0] = h2

    psum = jnp.sum(h2, axis=1, keepdims=True)
    psq = jnp.sum(h2 * h2, axis=1, keepdims=True)
    first = pl.program_id(1) == 0

    @pl.when(first)
    def _():
        s2_ref[0] = psum
        q2_ref[0] = psq

    @pl.when(jnp.logical_not(first))
    def _():
        s2_ref[0] += psum
        q2_ref[0] += psq


def _stage_c(h2_ref, s_ref, q_ref, gam_ref, bet_ref, out_ref, *, count):
    mean = jnp.sum(s_ref[...], axis=0) / count
    var = jnp.sum(q_ref[...], axis=0) / count - mean * mean
    scale = gam_ref[...] * jax.lax.rsqrt(var + 1e-5)
    shift = bet_ref[...] - mean * scale
    out_ref[0] = jnp.maximum(scale * h2_ref[0] + shift, 0.0)


def kernel(xyz1, xyz2, features1, features2, W1, g1, b1, W2, g2, b2):
    B, N1, _ = xyz1.shape
    N2 = xyz2.shape[1]
    C1 = features1.shape[1]
    C2 = features2.shape[1]
    CO1 = W1.shape[0]
    CO2 = W2.shape[0]
    TN = 4096
    NT = N1 // TN
    f32 = jnp.float32

    bf16 = jnp.bfloat16
    x1t = jnp.transpose(xyz1, (0, 2, 1)).astype(bf16)
    x2t = jnp.transpose(xyz2, (0, 2, 1)).astype(bf16)
    x1sq = jnp.sum(xyz1 ** 2, axis=-1)
    x2sq = jnp.sum(xyz2 ** 2, axis=-1)
    zer1 = jnp.zeros((B, 13, N1), bf16)
    zer2 = jnp.zeros((B, 13, N2), bf16)
    G1 = jnp.concatenate([x1t, zer1], axis=1)
    G2 = jnp.concatenate([x2t, zer2], axis=1)
    X1SQ = x1sq[:, None, :]
    X2SQ = x2sq[:, :, None]

    W1a = W1[:, :C1]
    W1b = W1[:, C1:]

    grid = (B, NT)
    params = pltpu.CompilerParams(
        dimension_semantics=("parallel", "arbitrary"))
    h1, s1, q1 = pl.pallas_call(
        functools.partial(_stage_a, n2=N2),
        grid=grid,
        in_specs=[
            pl.BlockSpec((1, 16, N2), lambda b, t: (b, 0, 0)),
            pl.BlockSpec((1, 16, TN), lambda b, t: (b, 0, t)),
            pl.BlockSpec((1, 1, TN), lambda b, t: (b, 0, t)),
            pl.BlockSpec((1, N2, 1), lambda b, t: (b, 0, 0)),
            pl.BlockSpec((1, C1, TN), lambda b, t: (b, 0, t)),
            pl.BlockSpec((1, C2, N2), lambda b, t: (b, 0, 0)),
            pl.BlockSpec((CO1, C1), lambda b, t: (0, 0)),
            pl.BlockSpec((CO1, C2), lambda b, t: (0, 0)),
        ],
        out_specs=[
            pl.BlockSpec((1, CO1, TN), lambda b, t: (b, 0, t)),
            pl.BlockSpec((1, CO1, 1), lambda b, t: (b, 0, 0)),
            pl.BlockSpec((1, CO1, 1), lambda b, t: (b, 0, 0)),
        ],
        out_shape=[
            jax.ShapeDtypeStruct((B, CO1, N1), f32),
            jax.ShapeDtypeStruct((B, CO1, 1), f32),
            jax.ShapeDtypeStruct((B, CO1, 1), f32),
        ],
        compiler_params=params,
    )(G2, G1, X1SQ, X2SQ, features1, features2, W1a, W1b)

    count = float(B * N1)
    h2, s2, q2 = pl.pallas_call(
        functools.partial(_stage_b, count=count),
        grid=grid,
        in_specs=[
            pl.BlockSpec((1, CO1, TN), lambda b, t: (b, 0, t)),
            pl.BlockSpec((B, CO1, 1), lambda b, t: (0, 0, 0)),
            pl.BlockSpec((B, CO1, 1), lambda b, t: (0, 0, 0)),
            pl.BlockSpec((CO1, 1), lambda b, t: (0, 0)),
            pl.BlockSpec((CO1, 1), lambda b, t: (0, 0)),
            pl.BlockSpec((CO2, CO1), lambda b, t: (0, 0)),
        ],
        out_specs=[
            pl.BlockSpec((1, CO2, TN), lambda b, t: (b, 0, t)),
            pl.BlockSpec((1, CO2, 1), lambda b, t: (b, 0, 0)),
            pl.BlockSpec((1, CO2, 1), lambda b, t: (b, 0, 0)),
        ],
        out_shape=[
            jax.ShapeDtypeStruct((B, CO2, N1), f32),
            jax.ShapeDtypeStruct((B, CO2, 1), f32),
            jax.ShapeDtypeStruct((B, CO2, 1), f32),
        ],
        compiler_params=params,
    )(h1, s1, q1, g1.reshape(CO1, 1), b1.reshape(CO1, 1), W2)

    out = pl.pallas_call(
        functools.partial(_stage_c, count=count),
        grid=grid,
        in_specs=[
            pl.BlockSpec((1, CO2, TN), lambda b, t: (b, 0, t)),
            pl.BlockSpec((B, CO2, 1), lambda b, t: (0, 0, 0)),
            pl.BlockSpec((B, CO2, 1), lambda b, t: (0, 0, 0)),
            pl.BlockSpec((CO2, 1), lambda b, t: (0, 0)),
            pl.BlockSpec((CO2, 1), lambda b, t: (0, 0)),
        ],
        out_specs=pl.BlockSpec((1, CO2, TN), lambda b, t: (b, 0, t)),
        out_shape=jax.ShapeDtypeStruct((B, CO2, N1), f32),
        compiler_params=params,
    )(h2, s2, q2, g2.reshape(CO2, 1), b2.reshape(CO2, 1))

    return out


# speedup vs baseline: 3.1663x; 1.0360x over previous
import functools

import jax
import jax.numpy as jnp
from jax.experimental import pallas as pl
from jax.experimental.pallas import tpu as pltpu


def _stage_a(g2_ref, g1_ref, x1sq_ref, x2sq_ref, f1_ref, f2_ref,
             w1a_ref, w1b_ref, h1_ref, s_ref, q_ref, *, n2):
    g2 = g2_ref[0]
    g1 = g1_ref[0]
    cross = jax.lax.dot_general(g2, g1, (((0,), (0,)), ((), ())),
                                preferred_element_type=jnp.float32)
    x1sq = x1sq_ref[0, 0, :]
    d2 = (x2sq_ref[0] + x1sq[None, :]) - 2.0 * cross

    iota = jax.lax.broadcasted_iota(jnp.int32, d2.shape, 0)
    cur = d2
    mins, amins = [], []
    for _ in range(3):
        m = jnp.min(cur, axis=0)
        am = jnp.min(jnp.where(cur == m[None, :], iota, n2), axis=0)
        mins.append(m)
        amins.append(am)
        cur = jnp.where(iota == am[None, :], jnp.float32(3e38), cur)

    ws = [1.0 / (jnp.sqrt(jnp.maximum(m, 0.0)) + 1e-8) for m in mins]
    norm = ws[0] + ws[1] + ws[2]
    ws = [w / norm for w in ws]

    st = jnp.where(iota == amins[0][None, :], ws[0][None, :], 0.0)
    st = st + jnp.where(iota == amins[1][None, :], ws[1][None, :], 0.0)
    st = st + jnp.where(iota == amins[2][None, :], ws[2][None, :], 0.0)

    interp = jnp.dot(f2_ref[0], st, preferred_element_type=jnp.float32)
    h1 = (jnp.dot(w1a_ref[...], f1_ref[0], preferred_element_type=jnp.float32)
          + jnp.dot(w1b_ref[...], interp, preferred_element_type=jnp.float32))
    h1_ref[0] = h1.astype(jnp.bfloat16)

    psum = jnp.sum(h1, axis=1, keepdims=True)
    psq = jnp.sum(h1 * h1, axis=1, keepdims=True)
    first = pl.program_id(1) == 0

    @pl.when(first)
    def _():
        s_ref[0] = psum
        q_ref[0] = psq

    @pl.when(jnp.logical_not(first))
    def _():
        s_ref[0] += psum
        q_ref[0] += psq


def _stage_b(h1_ref, s_ref, q_ref, gam_ref, bet_ref, w2_ref,
             h2_ref, s2_ref, q2_ref, *, count):
    mean = jnp.sum(s_ref[...], axis=0) / count
    var = jnp.sum(q_ref[...], axis=0) / count - mean * mean
    scale = gam_ref[...] * jax.lax.rsqrt(var + 1e-5)
    shift = bet_ref[...] - mean * scale
    act = jnp.maximum(scale * h1_ref[0].astype(jnp.float32) + shift, 0.0)
    h2 = jnp.dot(w2_ref[...], act, preferred_element_type=jnp.float32)
    h2_ref[0] = h2.astype(jnp.bfloat16)

    psum = jnp.sum(h2, axis=1, keepdims=True)
    psq = jnp.sum(h2 * h2, axis=1, keepdims=True)
    first = pl.program_id(1) == 0

    @pl.when(first)
    def _():
        s2_ref[0] = psum
        q2_ref[0] = psq

    @pl.when(jnp.logical_not(first))
    def _():
        s2_ref[0] += psum
        q2_ref[0] += psq


def _stage_c(h2_ref, s_ref, q_ref, gam_ref, bet_ref, out_ref, *, count):
    mean = jnp.sum(s_ref[...], axis=0) / count
    var = jnp.sum(q_ref[...], axis=0) / count - mean * mean
    scale = gam_ref[...] * jax.lax.rsqrt(var + 1e-5)
    shift = bet_ref[...] - mean * scale
    out_ref[0] = jnp.maximum(scale * h2_ref[0].astype(jnp.float32) + shift, 0.0)


def kernel(xyz1, xyz2, features1, features2, W1, g1, b1, W2, g2, b2):
    B, N1, _ = xyz1.shape
    N2 = xyz2.shape[1]
    C1 = features1.shape[1]
    C2 = features2.shape[1]
    CO1 = W1.shape[0]
    CO2 = W2.shape[0]
    TN = 4096
    NT = N1 // TN
    f32 = jnp.float32

    bf16 = jnp.bfloat16
    x1t = jnp.transpose(xyz1, (0, 2, 1)).astype(bf16)
    x2t = jnp.transpose(xyz2, (0, 2, 1)).astype(bf16)
    x1sq = jnp.sum(xyz1 ** 2, axis=-1)
    x2sq = jnp.sum(xyz2 ** 2, axis=-1)
    zer1 = jnp.zeros((B, 13, N1), bf16)
    zer2 = jnp.zeros((B, 13, N2), bf16)
    G1 = jnp.concatenate([x1t, zer1], axis=1)
    G2 = jnp.concatenate([x2t, zer2], axis=1)
    X1SQ = x1sq[:, None, :]
    X2SQ = x2sq[:, :, None]

    W1a = W1[:, :C1]
    W1b = W1[:, C1:]

    grid = (B, NT)
    params = pltpu.CompilerParams(
        dimension_semantics=("parallel", "arbitrary"))
    h1, s1, q1 = pl.pallas_call(
        functools.partial(_stage_a, n2=N2),
        grid=grid,
        in_specs=[
            pl.BlockSpec((1, 16, N2), lambda b, t: (b, 0, 0)),
            pl.BlockSpec((1, 16, TN), lambda b, t: (b, 0, t)),
            pl.BlockSpec((1, 1, TN), lambda b, t: (b, 0, t)),
            pl.BlockSpec((1, N2, 1), lambda b, t: (b, 0, 0)),
            pl.BlockSpec((1, C1, TN), lambda b, t: (b, 0, t)),
            pl.BlockSpec((1, C2, N2), lambda b, t: (b, 0, 0)),
            pl.BlockSpec((CO1, C1), lambda b, t: (0, 0)),
            pl.BlockSpec((CO1, C2), lambda b, t: (0, 0)),
        ],
        out_specs=[
            pl.BlockSpec((1, CO1, TN), lambda b, t: (b, 0, t)),
            pl.BlockSpec((1, CO1, 1), lambda b, t: (b, 0, 0)),
            pl.BlockSpec((1, CO1, 1), lambda b, t: (b, 0, 0)),
        ],
        out_shape=[
            jax.ShapeDtypeStruct((B, CO1, N1), jnp.bfloat16),
            jax.ShapeDtypeStruct((B, CO1, 1), f32),
            jax.ShapeDtypeStruct((B, CO1, 1), f32),
        ],
        compiler_params=params,
    )(G2, G1, X1SQ, X2SQ, features1, features2, W1a, W1b)

    count = float(B * N1)
    h2, s2, q2 = pl.pallas_call(
        functools.partial(_stage_b, count=count),
        grid=grid,
        in_specs=[
            pl.BlockSpec((1, CO1, TN), lambda b, t: (b, 0, t)),
            pl.BlockSpec((B, CO1, 1), lambda b, t: (0, 0, 0)),
            pl.BlockSpec((B, CO1, 1), lambda b, t: (0, 0, 0)),
            pl.BlockSpec((CO1, 1), lambda b, t: (0, 0)),
            pl.BlockSpec((CO1, 1), lambda b, t: (0, 0)),
            pl.BlockSpec((CO2, CO1), lambda b, t: (0, 0)),
        ],
        out_specs=[
            pl.BlockSpec((1, CO2, TN), lambda b, t: (b, 0, t)),
            pl.BlockSpec((1, CO2, 1), lambda b, t: (b, 0, 0)),
            pl.BlockSpec((1, CO2, 1), lambda b, t: (b, 0, 0)),
        ],
        out_shape=[
            jax.ShapeDtypeStruct((B, CO2, N1), jnp.bfloat16),
            jax.ShapeDtypeStruct((B, CO2, 1), f32),
            jax.ShapeDtypeStruct((B, CO2, 1), f32),
        ],
        compiler_params=params,
    )(h1, s1, q1, g1.reshape(CO1, 1), b1.reshape(CO1, 1), W2)

    out = pl.pallas_call(
        functools.partial(_stage_c, count=count),
        grid=grid,
        in_specs=[
            pl.BlockSpec((1, CO2, TN), lambda b, t: (b, 0, t)),
            pl.BlockSpec((B, CO2, 1), lambda b, t: (0, 0, 0)),
            pl.BlockSpec((B, CO2, 1), lambda b, t: (0, 0, 0)),
            pl.BlockSpec((CO2, 1), lambda b, t: (0, 0)),
            pl.BlockSpec((CO2, 1), lambda b, t: (0, 0)),
        ],
        out_specs=pl.BlockSpec((1, CO2, TN), lambda b, t: (b, 0, t)),
        out_shape=jax.ShapeDtypeStruct((B, CO2, N1), f32),
        compiler_params=params,
    )(h2, s2, q2, g2.reshape(CO2, 1), b2.reshape(CO2, 1))

    return out
